# baseline (speedup 1.0000x reference)
import functools

import jax
import jax.numpy as jnp
from jax import lax
from jax.experimental import pallas as pl
from jax.experimental.pallas import tpu as pltpu
from jax.experimental.pallas import tpu_sc as plsc

NC = 2
NS = 16
NW = NC * NS
LANES = 16
CHUNK = 128


TB = 16384


def _repack_split(V):
    nblk = -(-V // (2 * TB))
    return nblk * TB, nblk


def _make_repack(V, D):
    hsplit, nblk = _repack_split(V)
    last = -(-V // TB) - 1

    def body(a_ref, b_ref, out_ref):
        out_ref[...] = jnp.concatenate(
            [jnp.transpose(a_ref[...], (1, 0)),
             jnp.transpose(b_ref[...], (1, 0))], axis=1)

    return pl.pallas_call(
        body,
        grid=(nblk,),
        in_specs=[
            pl.BlockSpec((D, TB), lambda i: (0, i)),
            pl.BlockSpec((D, TB), lambda i: (0, jnp.minimum(i + nblk, last))),
        ],
        out_specs=pl.BlockSpec((TB, 2 * D), lambda i: (i, 0)),
        out_shape=jax.ShapeDtypeStruct((hsplit, 2 * D), jnp.float32),
    )


def _make_kernel(B, S, V, D):
    rows_total = B * S
    rows_per_w = rows_total // NW
    chunks = rows_per_w // CHUNK
    assert rows_per_w % CHUNK == 0 and rows_per_w % S == 0
    assert D == 64 and V % 2 == 0 and chunks % 2 == 0

    mesh = plsc.VectorSubcoreMesh(
        core_axis_name="c", subcore_axis_name="s",
        num_cores=NC, num_subcores=NS)

    @functools.partial(
        pl.kernel,
        out_type=jax.ShapeDtypeStruct((rows_total // 8, 8, D), jnp.float32),
        mesh=mesh,
        scratch_types=[
            pltpu.VMEM((2, CHUNK), jnp.int32),
            pltpu.VMEM((2, CHUNK, LANES), jnp.int32),
            pltpu.VMEM((2, CHUNK, 2 * D), jnp.float32),
            pltpu.VMEM((2, CHUNK // 8, 8, D), jnp.float32),
            pltpu.SemaphoreType.DMA,
            pltpu.SemaphoreType.DMA,
            pltpu.SemaphoreType.DMA,
            pltpu.SemaphoreType.DMA,
            pltpu.SemaphoreType.DMA,
            pltpu.SemaphoreType.DMA,
        ],
    )
    def k(idx2_hbm, msk_hbm, tab2_hbm, post_hbm, out_hbm,
          idx_v, msk_v, rows_v, res_v,
          gsem0, gsem1, osem0, osem1, xsem0, xsem1):
        gsems = (gsem0, gsem1)
        osems = (osem0, osem1)
        xsems = (xsem0, xsem1)
        wid = lax.axis_index("s") * NC + lax.axis_index("c")

        def start_gather(c, b):
            pltpu.sync_copy(idx2_hbm.at[wid, c], idx_v.at[b])
            pltpu.async_copy(tab2_hbm.at[idx_v.at[b]], rows_v.at[b], gsems[b])
            pltpu.async_copy(msk_hbm.at[wid, c], msk_v.at[b], xsems[b])
            pltpu.async_copy(post_hbm.at[c], res_v.at[b], xsems[b])

        def wait_gather(c, b):
            pltpu.make_async_copy(
                tab2_hbm.at[idx_v.at[b]], rows_v.at[b], gsems[b]).wait()
            pltpu.make_async_copy(
                msk_hbm.at[wid, c], msk_v.at[b], xsems[b]).wait()
            pltpu.make_async_copy(
                post_hbm.at[c], res_v.at[b], xsems[b]).wait()

        def out_slice(c):
            return out_hbm.at[pl.ds(wid * (rows_per_w // 8)
                                    + c * (CHUNK // 8), CHUNK // 8)]

        start_gather(0, 0)

        @pl.loop(0, chunks // 2)
        def _pair(cc):
            for b in range(2):
                c = 2 * cc + b

                @pl.when(c + 1 < chunks)
                def _():
                    @pl.when(c >= 1)
                    def _():
                        pltpu.make_async_copy(res_v.at[1 - b],
                                              out_slice(c - 1),
                                              osems[1 - b]).wait()
                    start_gather(c + 1, 1 - b)

                wait_gather(c, b)

                @pl.loop(0, CHUNK // 8, unroll=8)
                def _oblk(q):
                    for p in range(8):
                        r = 8 * q + p
                        m = msk_v[b, r, pl.ds(0, LANES)]
                        for j in range(D // LANES):
                            g0 = rows_v[b, r, pl.ds(16 * j, LANES)]
                            g1 = rows_v[b, r, pl.ds(D + 16 * j, LANES)]
                            g = jnp.where(m > 0, g1, g0)
                            res = g + res_v[b, q, p, pl.ds(16 * j, LANES)]
                            res_v[b, q, p, pl.ds(16 * j, LANES)] = res

                pltpu.async_copy(res_v.at[b], out_slice(c), osems[b])

        for b in range(2):
            pltpu.make_async_copy(res_v.at[b], out_slice(chunks - 2 + b),
                                  osems[b]).wait()

    return k


def kernel(x, token_table, pos_table):
    B, S = x.shape
    V, D = token_table.shape
    xf = x.reshape(-1).astype(jnp.int32)
    rows_total = B * S
    chunks = rows_total // NW // CHUNK
    V2, _ = _repack_split(V)
    hi = xf >= V2
    idx2 = jnp.where(hi, xf - V2, xf).reshape(NW, chunks, CHUNK)
    msk = jnp.broadcast_to(hi.astype(jnp.int32).reshape(NW, chunks, CHUNK, 1),
                           (NW, chunks, CHUNK, LANES))
    tabT = token_table.T
    tab2 = _make_repack(V, D)(tabT, tabT)
    reps = S // _gcd(S, CHUNK)
    tile0 = jnp.tile(pos_table, (CHUNK * reps // S, 1))
    tile0 = tile0.reshape(reps, CHUNK // 8, 8, D)
    post = jnp.tile(tile0, (chunks // reps, 1, 1, 1))
    k = _make_kernel(B, S, V, D)
    out = k(idx2, msk, tab2, post)
    return out.reshape(B, S, D)


def _gcd(a, b):
    while b:
        a, b = b, a % b
    return a


# speedup vs baseline: 1.1354x; 1.1354x over previous
import functools

import jax
import jax.numpy as jnp
from jax import lax
from jax.experimental import pallas as pl
from jax.experimental.pallas import tpu as pltpu
from jax.experimental.pallas import tpu_sc as plsc

NC = 2
NS = 16
NW = NC * NS
LANES = 16
CHUNK = 128


TB = 16384


def _repack_split(V):
    nblk = -(-V // (2 * TB))
    return nblk * TB, nblk


def _make_repack(V, D):
    hsplit, nblk = _repack_split(V)
    last = -(-V // TB) - 1

    def body(a_ref, b_ref, out_ref):
        out_ref[...] = jnp.concatenate(
            [jnp.transpose(a_ref[...], (1, 0)),
             jnp.transpose(b_ref[...], (1, 0))], axis=1)

    return pl.pallas_call(
        body,
        grid=(nblk,),
        in_specs=[
            pl.BlockSpec((D, TB), lambda i: (0, i)),
            pl.BlockSpec((D, TB), lambda i: (0, jnp.minimum(i + nblk, last))),
        ],
        out_specs=pl.BlockSpec((TB, 2 * D), lambda i: (i, 0)),
        out_shape=jax.ShapeDtypeStruct((hsplit, 2 * D), jnp.float32),
    )


def _make_kernel(B, S, V, D):
    rows_total = B * S
    rows_per_w = rows_total // NW
    chunks = rows_per_w // CHUNK
    assert rows_per_w % CHUNK == 0 and rows_per_w % S == 0
    assert D == 64 and V % 2 == 0 and chunks % 2 == 0

    mesh = plsc.VectorSubcoreMesh(
        core_axis_name="c", subcore_axis_name="s",
        num_cores=NC, num_subcores=NS)

    @functools.partial(
        pl.kernel,
        out_type=jax.ShapeDtypeStruct((rows_total // 8, 8, D), jnp.float32),
        mesh=mesh,
        scratch_types=[
            pltpu.VMEM((2, CHUNK), jnp.int32),
            pltpu.VMEM((2, CHUNK + LANES), jnp.float32),
            pltpu.VMEM((2, CHUNK, 2 * D), jnp.float32),
            pltpu.VMEM((2, CHUNK // 8, 8, D), jnp.float32),
            pltpu.SemaphoreType.DMA,
            pltpu.SemaphoreType.DMA,
            pltpu.SemaphoreType.DMA,
            pltpu.SemaphoreType.DMA,
            pltpu.SemaphoreType.DMA,
            pltpu.SemaphoreType.DMA,
        ],
    )
    def k(idx2_hbm, msk_hbm, tab2_hbm, post_hbm, out_hbm,
          idx_v, msk_v, rows_v, res_v,
          gsem0, gsem1, osem0, osem1, xsem0, xsem1):
        gsems = (gsem0, gsem1)
        osems = (osem0, osem1)
        xsems = (xsem0, xsem1)
        wid = lax.axis_index("s") * NC + lax.axis_index("c")

        def start_gather(c, b):
            pltpu.sync_copy(idx2_hbm.at[wid, c], idx_v.at[b])
            pltpu.async_copy(tab2_hbm.at[idx_v.at[b]], rows_v.at[b], gsems[b])
            pltpu.async_copy(msk_hbm.at[wid, c],
                             msk_v.at[b, pl.ds(0, CHUNK)], xsems[b])
            pltpu.async_copy(post_hbm.at[c], res_v.at[b], xsems[b])

        def wait_gather(c, b):
            pltpu.make_async_copy(
                tab2_hbm.at[idx_v.at[b]], rows_v.at[b], gsems[b]).wait()
            pltpu.make_async_copy(
                msk_hbm.at[wid, c], msk_v.at[b, pl.ds(0, CHUNK)],
                xsems[b]).wait()
            pltpu.make_async_copy(
                post_hbm.at[c], res_v.at[b], xsems[b]).wait()

        def out_slice(c):
            return out_hbm.at[pl.ds(wid * (rows_per_w // 8)
                                    + c * (CHUNK // 8), CHUNK // 8)]

        start_gather(0, 0)

        @pl.loop(0, chunks // 2)
        def _pair(cc):
            for b in range(2):
                c = 2 * cc + b

                @pl.when(c + 1 < chunks)
                def _():
                    @pl.when(c >= 1)
                    def _():
                        pltpu.make_async_copy(res_v.at[1 - b],
                                              out_slice(c - 1),
                                              osems[1 - b]).wait()
                    start_gather(c + 1, 1 - b)

                wait_gather(c, b)

                @pl.loop(0, CHUNK // LANES, unroll=2)
                def _grp(gq):
                    mv0 = msk_v[b, pl.ds(LANES * gq, LANES)]
                    for p in range(LANES):
                        r = LANES * gq + p
                        q3 = 2 * gq + (p // 8)
                        rp = p % 8
                        m = lax.gather(
                            mv0,
                            jnp.full((LANES, 1), p, jnp.int32),
                            lax.GatherDimensionNumbers(
                                offset_dims=(), collapsed_slice_dims=(0,),
                                start_index_map=(0,)),
                            (1,), mode=lax.GatherScatterMode.PROMISE_IN_BOUNDS)
                        for j in range(D // LANES):
                            g0 = rows_v[b, r, pl.ds(16 * j, LANES)]
                            g1 = rows_v[b, r, pl.ds(D + 16 * j, LANES)]
                            g = g0 + m * (g1 - g0)
                            res = g + res_v[b, q3, rp, pl.ds(16 * j, LANES)]
                            res_v[b, q3, rp, pl.ds(16 * j, LANES)] = res

                pltpu.async_copy(res_v.at[b], out_slice(c), osems[b])

        for b in range(2):
            pltpu.make_async_copy(res_v.at[b], out_slice(chunks - 2 + b),
                                  osems[b]).wait()

    return k


def kernel(x, token_table, pos_table):
    B, S = x.shape
    V, D = token_table.shape
    xf = x.reshape(-1).astype(jnp.int32)
    rows_total = B * S
    chunks = rows_total // NW // CHUNK
    V2, _ = _repack_split(V)
    hi = xf >= V2
    idx2 = jnp.where(hi, xf - V2, xf).reshape(NW, chunks, CHUNK)
    msk = hi.astype(jnp.float32).reshape(NW, chunks, CHUNK)
    tabT = token_table.T
    tab2 = _make_repack(V, D)(tabT, tabT)
    reps = S // _gcd(S, CHUNK)
    tile0 = jnp.tile(pos_table, (CHUNK * reps // S, 1))
    tile0 = tile0.reshape(reps, CHUNK // 8, 8, D)
    post = jnp.tile(tile0, (chunks // reps, 1, 1, 1))
    k = _make_kernel(B, S, V, D)
    out = k(idx2, msk, tab2, post)
    return out.reshape(B, S, D)


def _gcd(a, b):
    while b:
        a, b = b, a % b
    return a
